# Initial kernel scaffold; baseline (speedup 1.0000x reference)
import functools

import jax
import jax.numpy as jnp
import numpy as np
from jax import lax
from jax.experimental import pallas as pl
from jax.experimental.pallas import tpu as pltpu
from jax.experimental.pallas import tpu_sc as plsc

_MASK_FREQ = 0.15
_TOTAL_TOKENS = 32768
_D_FEAT = 512

_KEPT = np.asarray(
    jax.random.uniform(jax.random.key(42), (_TOTAL_TOKENS,), dtype=jnp.float32)
    > _MASK_FREQ
)
_N_KEPT_ROWS = int(_KEPT.sum())
_IDX = np.nonzero(_KEPT)[0].astype(np.int32)

_NW = 32
_NC = 2
_CH = 64
_NCHUNKS = -(-_N_KEPT_ROWS // _CH)
_TAIL = _N_KEPT_ROWS - (_NCHUNKS - 1) * _CH
_ITERS = -(-_NCHUNKS // _NW)

_IDX_PAD = np.zeros((_NCHUNKS * _CH,), np.int32)
_IDX_PAD[:_N_KEPT_ROWS] = _IDX
_IDX_PAD_J = jnp.asarray(_IDX_PAD)


@functools.partial(
    pl.kernel,
    mesh=plsc.VectorSubcoreMesh(core_axis_name="c", subcore_axis_name="s"),
    out_type=jax.ShapeDtypeStruct((_N_KEPT_ROWS, _D_FEAT), jnp.float32),
    scratch_types=[
        pltpu.VMEM((_CH,), jnp.int32),
        pltpu.VMEM((_CH, _D_FEAT), jnp.float32),
        pltpu.SemaphoreType.DMA,
    ],
)
def _sc_gather(feat_hbm, idx_hbm, out_hbm, idx_v, rows_v, sem):
    w = lax.axis_index("s") * _NC + lax.axis_index("c")

    def body(i, _):
        c = w + i * _NW

        @pl.when(c < _NCHUNKS)
        def _chunk():
            pltpu.sync_copy(idx_hbm.at[pl.ds(c * _CH, _CH)], idx_v)
            pltpu.async_copy(feat_hbm.at[idx_v], rows_v, sem).wait()

            @pl.when(c < _NCHUNKS - 1)
            def _full():
                pltpu.sync_copy(rows_v, out_hbm.at[pl.ds(c * _CH, _CH)])

            @pl.when(c == _NCHUNKS - 1)
            def _tail():
                pltpu.sync_copy(
                    rows_v.at[pl.ds(0, _TAIL)],
                    out_hbm.at[pl.ds(c * _CH, _TAIL)],
                )

        return ()

    lax.fori_loop(0, _ITERS, body, ())


def kernel(feature):
    return _sc_gather(feature, _IDX_PAD_J)


# speedup vs baseline: 2.7188x; 2.7188x over previous
import functools

import jax
import jax.numpy as jnp
import numpy as np
from jax import lax
from jax.experimental import pallas as pl
from jax.experimental.pallas import tpu as pltpu
from jax.experimental.pallas import tpu_sc as plsc

_MASK_FREQ = 0.15
_TOTAL_TOKENS = 32768
_D_FEAT = 512


def _rotl32(x, r):
    return ((x << np.uint32(r)) | (x >> np.uint32(32 - r))).astype(np.uint32)


def _threefry2x32(k0, k1, x0, x1):
    rotations = ((13, 15, 26, 6), (17, 29, 16, 24))
    ks = (
        np.uint32(k0),
        np.uint32(k1),
        np.uint32(k0) ^ np.uint32(k1) ^ np.uint32(0x1BD11BDA),
    )
    x0 = (x0 + ks[0]).astype(np.uint32)
    x1 = (x1 + ks[1]).astype(np.uint32)
    for d in range(5):
        for rot in rotations[d % 2]:
            x0 = (x0 + x1).astype(np.uint32)
            x1 = _rotl32(x1, rot)
            x1 = (x1 ^ x0).astype(np.uint32)
        x0 = (x0 + ks[(d + 1) % 3]).astype(np.uint32)
        x1 = (x1 + ks[(d + 2) % 3] + np.uint32(d + 1)).astype(np.uint32)
    return x0, x1


def _uniform_key42(n):
    lo = np.arange(n, dtype=np.uint32)
    hi = np.zeros(n, np.uint32)
    x0, x1 = _threefry2x32(0, 42, hi, lo)
    bits = (x0 ^ x1).astype(np.uint32)
    return ((bits >> np.uint32(9)) | np.uint32(0x3F800000)).view(np.float32) - np.float32(1.0)


_KEPT = _uniform_key42(_TOTAL_TOKENS) > _MASK_FREQ
_N_KEPT_ROWS = int(_KEPT.sum())
_IDX = np.nonzero(_KEPT)[0].astype(np.int32)

_NW = 32
_NC = 2
_CH = 64
_NCHUNKS = -(-_N_KEPT_ROWS // _CH)
_TAIL = _N_KEPT_ROWS - (_NCHUNKS - 1) * _CH
_ITERS = -(-_NCHUNKS // _NW)

_IDX_PAD = np.full((_NCHUNKS * _CH,), _IDX[-1], np.int32)
_IDX_PAD[:_N_KEPT_ROWS] = _IDX

_DST_TAIL = np.minimum(
    np.arange((_NCHUNKS - 1) * _CH, _NCHUNKS * _CH), _N_KEPT_ROWS - 1
).astype(np.int32)


@functools.cache
def _build_sc_gather():
    mesh = plsc.VectorSubcoreMesh(core_axis_name="c", subcore_axis_name="s")

    @functools.partial(
        pl.kernel,
        mesh=mesh,
        out_type=jax.ShapeDtypeStruct((_N_KEPT_ROWS, _D_FEAT), jnp.float32),
        scratch_types=[
            pltpu.VMEM((_CH,), jnp.int32),
            pltpu.VMEM((_CH,), jnp.int32),
            pltpu.VMEM((_CH, _D_FEAT), jnp.float32),
            pltpu.SemaphoreType.DMA,
        ],
    )
    def _sc_gather(feat_hbm, idx_hbm, dst_hbm, out_hbm, idx_v, dst_v, rows_v, sem):
        w = lax.axis_index("s") * _NC + lax.axis_index("c")

        def body(i, _):
            c = w + i * _NW

            @pl.when(c < _NCHUNKS)
            def _chunk():
                pltpu.sync_copy(idx_hbm.at[pl.ds(c * _CH, _CH)], idx_v)
                pltpu.async_copy(feat_hbm.at[idx_v], rows_v, sem).wait()

                @pl.when(c < _NCHUNKS - 1)
                def _full():
                    pltpu.sync_copy(rows_v, out_hbm.at[pl.ds(c * _CH, _CH)])

                @pl.when(c == _NCHUNKS - 1)
                def _tail():
                    pltpu.sync_copy(dst_hbm, dst_v)
                    pltpu.async_copy(rows_v, out_hbm.at[dst_v], sem).wait()

            return ()

        lax.fori_loop(0, _ITERS, body, ())

    return _sc_gather


def kernel(feature):
    return _build_sc_gather()(feature, jnp.asarray(_IDX_PAD), jnp.asarray(_DST_TAIL))


# speedup vs baseline: 3.1884x; 1.1727x over previous
import functools

import jax
import jax.numpy as jnp
import numpy as np
from jax import lax
from jax.experimental import pallas as pl
from jax.experimental.pallas import tpu as pltpu
from jax.experimental.pallas import tpu_sc as plsc

_MASK_FREQ = 0.15
_TOTAL_TOKENS = 32768
_D_FEAT = 512


def _rotl32(x, r):
    return ((x << np.uint32(r)) | (x >> np.uint32(32 - r))).astype(np.uint32)


def _threefry2x32(k0, k1, x0, x1):
    rotations = ((13, 15, 26, 6), (17, 29, 16, 24))
    ks = (
        np.uint32(k0),
        np.uint32(k1),
        np.uint32(k0) ^ np.uint32(k1) ^ np.uint32(0x1BD11BDA),
    )
    x0 = (x0 + ks[0]).astype(np.uint32)
    x1 = (x1 + ks[1]).astype(np.uint32)
    for d in range(5):
        for rot in rotations[d % 2]:
            x0 = (x0 + x1).astype(np.uint32)
            x1 = _rotl32(x1, rot)
            x1 = (x1 ^ x0).astype(np.uint32)
        x0 = (x0 + ks[(d + 1) % 3]).astype(np.uint32)
        x1 = (x1 + ks[(d + 2) % 3] + np.uint32(d + 1)).astype(np.uint32)
    return x0, x1


def _uniform_key42(n):
    lo = np.arange(n, dtype=np.uint32)
    hi = np.zeros(n, np.uint32)
    x0, x1 = _threefry2x32(0, 42, hi, lo)
    bits = (x0 ^ x1).astype(np.uint32)
    return ((bits >> np.uint32(9)) | np.uint32(0x3F800000)).view(np.float32) - np.float32(1.0)


_KEPT = _uniform_key42(_TOTAL_TOKENS) > _MASK_FREQ
_N_KEPT_ROWS = int(_KEPT.sum())
_IDX = np.nonzero(_KEPT)[0].astype(np.int32)

_NW = 32
_NC = 2
_CH = 64
_NCHUNKS = -(-_N_KEPT_ROWS // _CH)
_TAIL = _N_KEPT_ROWS - (_NCHUNKS - 1) * _CH
_ITERS = -(-_NCHUNKS // _NW)

_IDX_PAD = np.full((_NCHUNKS * _CH,), _IDX[-1], np.int32)
_IDX_PAD[:_N_KEPT_ROWS] = _IDX

_DST_TAIL = np.minimum(
    np.arange((_NCHUNKS - 1) * _CH, _NCHUNKS * _CH), _N_KEPT_ROWS - 1
).astype(np.int32)


@functools.cache
def _build_sc_gather():
    mesh = plsc.VectorSubcoreMesh(core_axis_name="c", subcore_axis_name="s")

    @functools.partial(
        pl.kernel,
        mesh=mesh,
        out_type=jax.ShapeDtypeStruct((_N_KEPT_ROWS, _D_FEAT), jnp.float32),
        scratch_types=[
            pltpu.VMEM((2, _CH), jnp.int32),
            pltpu.VMEM((_CH,), jnp.int32),
            pltpu.VMEM((_CH, _D_FEAT), jnp.float32),
            pltpu.VMEM((_CH, _D_FEAT), jnp.float32),
            pltpu.SemaphoreType.DMA,
            pltpu.SemaphoreType.DMA,
            pltpu.SemaphoreType.DMA,
            pltpu.SemaphoreType.DMA,
        ],
    )
    def _sc_gather(feat_hbm, idx_hbm, dst_hbm, out_hbm,
                   idx_v, dst_v, buf0, buf1, g0, g1, s0, s1):
        w = lax.axis_index("s") * _NC + lax.axis_index("c")
        bufs = (buf0, buf1)
        gsems = (g0, g1)
        ssems = (s0, s1)

        def start_gather(i):
            b = i % 2
            c = w + i * _NW

            @pl.when(c < _NCHUNKS)
            def _():
                pltpu.sync_copy(idx_hbm.at[pl.ds(c * _CH, _CH)], idx_v.at[b])
                pltpu.async_copy(feat_hbm.at[idx_v.at[b]], bufs[b], gsems[b])

        def finish_chunk(i):
            b = i % 2
            c = w + i * _NW

            @pl.when(c < _NCHUNKS)
            def _():
                pltpu.make_async_copy(
                    feat_hbm.at[idx_v.at[b]], bufs[b], gsems[b]
                ).wait()

                @pl.when(c < _NCHUNKS - 1)
                def _full():
                    pltpu.async_copy(
                        bufs[b], out_hbm.at[pl.ds(c * _CH, _CH)], ssems[b]
                    )

                @pl.when(c == _NCHUNKS - 1)
                def _tail():
                    pltpu.sync_copy(dst_hbm, dst_v)
                    pltpu.async_copy(bufs[b], out_hbm.at[dst_v], ssems[b])

        def wait_store(i):
            b = i % 2
            c = w + i * _NW

            @pl.when(c < _NCHUNKS - 1)
            def _():
                pltpu.make_async_copy(
                    bufs[b], out_hbm.at[pl.ds(c * _CH, _CH)], ssems[b]
                ).wait()

            @pl.when(c == _NCHUNKS - 1)
            def _():
                pltpu.make_async_copy(bufs[b], out_hbm.at[dst_v], ssems[b]).wait()

        start_gather(0)
        for i in range(_ITERS):
            if i + 1 < _ITERS:
                if i - 1 >= 0:
                    wait_store(i - 1)
                start_gather(i + 1)
            finish_chunk(i)
        for i in range(max(0, _ITERS - 2), _ITERS):
            wait_store(i)

    return _sc_gather


def kernel(feature):
    return _build_sc_gather()(feature, jnp.asarray(_IDX_PAD), jnp.asarray(_DST_TAIL))
